# baseline (speedup 1.0000x reference)
import jax
import jax.numpy as jnp
from jax import lax
from jax.experimental import pallas as pl
from jax.experimental.pallas import tpu as pltpu
from jax.experimental.pallas import tpu_sc as plsc

NC = 2
NS = 16
CH = 125

F32 = jnp.float32


def _make_spmm(n, d, e, with_deg):
    dh = d // NC
    per_tile = e // NS
    nch = per_tile // CH
    NBUF = 4 if with_deg else 5
    assert per_tile * NS == e and nch * CH == per_tile and nch % NBUF == 0
    npad = ((n + NS * 8 - 1) // (NS * 8)) * (NS * 8)
    rows_per_tile = npad // NS
    WBC = min(rows_per_tile, 80)
    wb_chunks = []
    off = 0
    while off < rows_per_tile:
        sz = min(WBC, rows_per_tile - off)
        wb_chunks.append((off, sz))
        off += sz

    mesh = plsc.VectorSubcoreMesh(core_axis_name="c", subcore_axis_name="s")

    out_type = [jax.ShapeDtypeStruct((NC, npad, dh), F32)]
    scratch = [
        pltpu.VMEM_SHARED((npad, dh), F32),
        pltpu.VMEM((nch, CH), jnp.int32),
        pltpu.VMEM((nch, CH), jnp.int32),
        pltpu.VMEM((WBC, dh), F32),
        pltpu.SemaphoreType.DMA,
    ]
    scratch += [pltpu.VMEM((CH, dh), F32)] * NBUF
    scratch += [pltpu.SemaphoreType.DMA] * (2 * NBUF)
    if with_deg:
        out_type.append(jax.ShapeDtypeStruct((NC, npad, 8), F32))
        scratch += [
            pltpu.VMEM_SHARED((npad, 8), F32),
            pltpu.VMEM((WBC, 8), F32),
            pltpu.VMEM((CH, 8), F32),
        ]
        scratch += [pltpu.SemaphoreType.DMA] * NBUF

    def body(z_hbm, src_hbm, dst_hbm, zeros_d_hbm, zeros16_hbm, *rest):
        if with_deg:
            (out_hbm, degout_hbm, acc_sh, src_all, dst_all, wb_v, semp,
             *bufs) = rest
            (deg_sh, wbd_v, ones_v, *semd) = bufs[3 * NBUF:]
            bufs = bufs[:3 * NBUF]
        else:
            (out_hbm, acc_sh, src_all, dst_all, wb_v, semp, *bufs) = rest
        rows = bufs[:NBUF]
        semg = bufs[NBUF:2 * NBUF]
        sems = bufs[2 * NBUF:3 * NBUF]
        c = lax.axis_index("c")
        s = lax.axis_index("s")
        row0 = s * rows_per_tile
        is_deg = with_deg

        pltpu.async_copy(src_hbm.at[s], src_all, semp)
        pltpu.async_copy(dst_hbm.at[s], dst_all, semp)
        pltpu.sync_copy(zeros_d_hbm.at[pl.ds(0, WBC)], wb_v)
        for off, sz in wb_chunks:
            pltpu.sync_copy(wb_v.at[pl.ds(0, sz)],
                            acc_sh.at[pl.ds(row0 + off, sz)])
        if is_deg:
            pltpu.sync_copy(zeros16_hbm.at[pl.ds(0, WBC)], wbd_v)
            for off, sz in wb_chunks:
                pltpu.sync_copy(wbd_v.at[pl.ds(0, sz)],
                                deg_sh.at[pl.ds(row0 + off, sz)])

            def ones_body(i, carry):
                ones_v[i, :] = jnp.ones((8,), F32)
                return carry
            lax.fori_loop(0, CH, ones_body, 0)
        pltpu.make_async_copy(src_hbm.at[s], src_all, semp).wait()
        pltpu.make_async_copy(dst_hbm.at[s], dst_all, semp).wait()

        zc = z_hbm.at[c]

        def start_gather(i, b):
            pltpu.async_copy(zc.at[src_all.at[i]], rows[b], semg[b])

        def wait_gather(b):
            pltpu.make_async_copy(zc.at[src_all.at[0]], rows[b],
                                  semg[b]).wait()

        def start_scatter(i, b):
            pltpu.async_copy(rows[b], acc_sh.at[dst_all.at[i]], sems[b],
                             add=True)
            if is_deg:
                @pl.when(c == b % 2)
                def _():
                    pltpu.async_copy(ones_v, deg_sh.at[dst_all.at[i]],
                                     semd[b], add=True)

        def wait_scatter(b):
            pltpu.make_async_copy(rows[b], acc_sh.at[dst_all.at[0]],
                                  sems[b]).wait()
            if is_deg:
                @pl.when(c == b % 2)
                def _():
                    pltpu.make_async_copy(ones_v,
                                          deg_sh.at[dst_all.at[0]],
                                          semd[b]).wait()

        for b in range(NBUF):
            start_gather(b, b)

        plsc.subcore_barrier()

        ngrp = nch // NBUF

        def grp(g, carry):
            for b in range(NBUF):
                wait_gather(b)
                start_scatter(NBUF * g + b, b)
            for b in range(NBUF):
                @pl.when(NBUF * (g + 1) + b < nch)
                def _(b=b):
                    wait_scatter(b)
                    start_gather(NBUF * (g + 1) + b, b)
            return carry
        lax.fori_loop(0, ngrp, grp, 0)

        for b in range(NBUF):
            wait_scatter(b)

        plsc.subcore_barrier()

        stg = [wb_v, rows[0]]
        nwb = len(wb_chunks)
        for k, (off, sz) in enumerate(wb_chunks):
            buf = stg[k % 2]
            sem = semg[k % 2]
            if k >= 2:
                poff, psz = wb_chunks[k - 2]
                pltpu.make_async_copy(
                    buf.at[pl.ds(0, psz)],
                    out_hbm.at[c, pl.ds(row0 + poff, psz)], sem).wait()
            pltpu.sync_copy(acc_sh.at[pl.ds(row0 + off, sz)],
                            buf.at[pl.ds(0, sz)])
            pltpu.async_copy(buf.at[pl.ds(0, sz)],
                             out_hbm.at[c, pl.ds(row0 + off, sz)], sem)
        for k in range(max(0, nwb - 2), nwb):
            off, sz = wb_chunks[k]
            pltpu.make_async_copy(
                stg[k % 2].at[pl.ds(0, sz)],
                out_hbm.at[c, pl.ds(row0 + off, sz)], semg[k % 2]).wait()
        if is_deg:
            for off, sz in wb_chunks:
                pltpu.sync_copy(deg_sh.at[pl.ds(row0 + off, sz)],
                                wbd_v.at[pl.ds(0, sz)])
                pltpu.sync_copy(wbd_v.at[pl.ds(0, sz)],
                                degout_hbm.at[c, pl.ds(row0 + off, sz)])

    return pl.kernel(
        body, out_type=out_type, mesh=mesh, scratch_types=scratch,
        compiler_params=pltpu.CompilerParams(use_tc_tiling_on_sc=False))


_RB = 10000


def _row_spec(d):
    return pl.BlockSpec((_RB, d), lambda b: (b, 0))


def _half_spec(dh):
    return pl.BlockSpec((NC, _RB, dh), lambda b: (0, b, 0))


def _full_spec(shape):
    nd = len(shape)
    return pl.BlockSpec(shape, lambda b: (0,) * nd)


def _split(t, z_ref):
    dh = t.shape[-1] // 2
    z_ref[0] = t[:, :dh]
    z_ref[1] = t[:, dh:]


def _tc1_body(x_ref, w1l_ref, z1_ref):
    _split(jnp.dot(x_ref[...], w1l_ref[...], preferred_element_type=F32),
           z1_ref)


def _tc2_body(p_ref, deg_ref, x_ref, w1r_ref, b1l_ref, w2l_ref,
              a1_ref, z2_ref, inv_ref):
    agg = jnp.concatenate([p_ref[0], p_ref[1]], axis=-1)
    deg = deg_ref[0, :, :1] + deg_ref[1, :, :1]
    inv = 1.0 / jnp.maximum(deg, 1.0)
    inv_ref[...] = jnp.broadcast_to(inv, inv_ref.shape)
    r1 = jnp.dot(x_ref[...], w1r_ref[...], preferred_element_type=F32)
    h1 = jnp.maximum(agg * inv + b1l_ref[...] + r1, 0.0)
    a1_ref[...] = h1
    _split(jnp.dot(h1, w2l_ref[...], preferred_element_type=F32), z2_ref)


def _tc3_body(p_ref, inv_ref, a1_ref, x_ref, w2r_ref, b2l_ref,
              gamma_ref, beta_ref, w3l_ref, a2_ref, z3_ref):
    agg = jnp.concatenate([p_ref[0], p_ref[1]], axis=-1)
    inv = inv_ref[:, :1]
    r2 = jnp.dot(a1_ref[...], w2r_ref[...], preferred_element_type=F32)
    t = agg * inv + b2l_ref[...] + r2
    gs = gamma_ref[...] * (1.0 / jnp.sqrt(1.0 + 1e-5))
    t = t * gs + beta_ref[...]
    h2 = jnp.maximum(t, 0.0) + x_ref[...]
    a2_ref[...] = h2
    _split(jnp.dot(h2, w3l_ref[...], preferred_element_type=F32), z3_ref)


def _tc4_body(p_ref, inv_ref, a2_ref, w3r_ref, b3l_ref, logp_ref, out_ref):
    agg = jnp.concatenate([p_ref[0], p_ref[1]], axis=-1)
    inv = inv_ref[:, :1]
    r3 = jnp.dot(a2_ref[...], w3r_ref[...], preferred_element_type=F32)
    out = agg * inv + b3l_ref[...] + r3
    out_ref[...] = out
    m = jnp.max(out, axis=-1, keepdims=True)
    ex = jnp.exp(out - m)
    lse = jnp.log(jnp.sum(ex, axis=-1, keepdims=True)) + m
    logp_ref[...] = out - lse


def kernel(x, edge_index, W1l, W1r, W2l, W2r, W3l, W3r, b1l, b2l, b3l,
           gamma, beta):
    n, d = x.shape
    dh = d // NC
    e = edge_index.shape[1]
    grid = (n // _RB,)

    spmm_deg = _make_spmm(n, d, e, with_deg=True)
    spmm = _make_spmm(n, d, e, with_deg=False)

    npad = ((n + NS * 8 - 1) // (NS * 8)) * (NS * 8)
    wbz = min(npad // NS, 80)
    zeros_d = jnp.zeros((wbz, dh), F32)
    zeros16 = jnp.zeros((wbz, 8), F32)
    b1l2 = b1l.reshape(1, d)
    b2l2 = b2l.reshape(1, d)
    b3l2 = b3l.reshape(1, d)
    gamma2 = gamma.reshape(1, d)
    beta2 = beta.reshape(1, d)
    ei = edge_index.astype(jnp.int32)
    nch = e // NS // CH
    src = ei[0].reshape(NS, nch, CH)
    dst = ei[1].reshape(NS, nch, CH)

    zsplit_shape = jax.ShapeDtypeStruct((NC, npad, dh), F32)

    z1 = pl.pallas_call(
        _tc1_body, grid=grid,
        in_specs=[_row_spec(d), _full_spec((d, d))],
        out_specs=_half_spec(dh),
        out_shape=zsplit_shape,
    )(x, W1l)
    p1, deg = spmm_deg(z1, src, dst, zeros_d, zeros16)
    a1, z2, inv16 = pl.pallas_call(
        _tc2_body, grid=grid,
        in_specs=[_half_spec(dh), _half_spec(8), _row_spec(d),
                  _full_spec((d, d)), _full_spec((1, d)), _full_spec((d, d))],
        out_specs=[_row_spec(d), _half_spec(dh), _row_spec(8)],
        out_shape=[jax.ShapeDtypeStruct((n, d), F32),
                   zsplit_shape,
                   jax.ShapeDtypeStruct((n, 8), F32)],
    )(p1, deg, x, W1r, b1l2, W2l)

    (p2,) = spmm(z2, src, dst, zeros_d, zeros16)
    a2, z3 = pl.pallas_call(
        _tc3_body, grid=grid,
        in_specs=[_half_spec(dh), _row_spec(8), _row_spec(d), _row_spec(d),
                  _full_spec((d, d)), _full_spec((1, d)), _full_spec((1, d)),
                  _full_spec((1, d)), _full_spec((d, d))],
        out_specs=[_row_spec(d), _half_spec(dh)],
        out_shape=[jax.ShapeDtypeStruct((n, d), F32), zsplit_shape],
    )(p2, inv16, a1, x, W2r, b2l2, gamma2, beta2, W3l)

    (p3,) = spmm(z3, src, dst, zeros_d, zeros16)
    logp, out = pl.pallas_call(
        _tc4_body, grid=grid,
        in_specs=[_half_spec(dh), _row_spec(8), _row_spec(d),
                  _full_spec((d, d)), _full_spec((1, d))],
        out_specs=[_row_spec(d), _row_spec(d)],
        out_shape=[jax.ShapeDtypeStruct((n, d), F32),
                   jax.ShapeDtypeStruct((n, d), F32)],
    )(p3, inv16, a2, W3r, b3l2)

    return (logp, a1, a2, out)


# speedup vs baseline: 1.0237x; 1.0237x over previous
import jax
import jax.numpy as jnp
from jax import lax
from jax.experimental import pallas as pl
from jax.experimental.pallas import tpu as pltpu
from jax.experimental.pallas import tpu_sc as plsc

NC = 2
NS = 16
CH = 125

F32 = jnp.float32


def _make_spmm(n, d, e, with_deg):
    dh = d // NC
    per_tile = e // NS
    nch = per_tile // CH
    NBUF = 4 if with_deg else 5
    assert per_tile * NS == e and nch * CH == per_tile and nch % NBUF == 0
    npad = ((n + NS * 8 - 1) // (NS * 8)) * (NS * 8)
    rows_per_tile = npad // NS
    WBC = min(rows_per_tile, 80)
    wb_chunks = []
    off = 0
    while off < rows_per_tile:
        sz = min(WBC, rows_per_tile - off)
        wb_chunks.append((off, sz))
        off += sz

    mesh = plsc.VectorSubcoreMesh(core_axis_name="c", subcore_axis_name="s")

    out_type = [jax.ShapeDtypeStruct((NC, npad, dh), F32)]
    scratch = [
        pltpu.VMEM_SHARED((npad, dh), F32),
        pltpu.VMEM((nch, CH), jnp.int32),
        pltpu.VMEM((nch, CH), jnp.int32),
        pltpu.VMEM((WBC, dh), F32),
        pltpu.SemaphoreType.DMA,
    ]
    scratch += [pltpu.VMEM((CH, dh), F32)] * NBUF
    scratch += [pltpu.SemaphoreType.DMA] * (2 * NBUF)
    if with_deg:
        out_type.append(jax.ShapeDtypeStruct((NC, npad, 8), F32))
        scratch += [
            pltpu.VMEM_SHARED((npad, 8), F32),
            pltpu.VMEM((WBC, 8), F32),
            pltpu.VMEM((CH, 8), F32),
        ]
        scratch += [pltpu.SemaphoreType.DMA] * NBUF

    def body(z_hbm, src_hbm, dst_hbm, zeros_d_hbm, zeros16_hbm, *rest):
        if with_deg:
            (out_hbm, degout_hbm, acc_sh, src_all, dst_all, wb_v, semp,
             *bufs) = rest
            (deg_sh, wbd_v, ones_v, *semd) = bufs[3 * NBUF:]
            bufs = bufs[:3 * NBUF]
        else:
            (out_hbm, acc_sh, src_all, dst_all, wb_v, semp, *bufs) = rest
        rows = bufs[:NBUF]
        semg = bufs[NBUF:2 * NBUF]
        sems = bufs[2 * NBUF:3 * NBUF]
        c = lax.axis_index("c")
        s = lax.axis_index("s")
        row0 = s * rows_per_tile
        is_deg = with_deg

        pltpu.async_copy(src_hbm.at[s], src_all, semp)
        pltpu.async_copy(dst_hbm.at[s], dst_all, semp)
        pltpu.sync_copy(zeros_d_hbm.at[pl.ds(0, WBC)], wb_v)
        for off, sz in wb_chunks:
            pltpu.sync_copy(wb_v.at[pl.ds(0, sz)],
                            acc_sh.at[pl.ds(row0 + off, sz)])
        if is_deg:
            pltpu.sync_copy(zeros16_hbm.at[pl.ds(0, WBC)], wbd_v)
            for off, sz in wb_chunks:
                pltpu.sync_copy(wbd_v.at[pl.ds(0, sz)],
                                deg_sh.at[pl.ds(row0 + off, sz)])

            def ones_body(i, carry):
                ones_v[i, :] = jnp.ones((8,), F32)
                return carry
            lax.fori_loop(0, CH, ones_body, 0)
        pltpu.make_async_copy(src_hbm.at[s], src_all, semp).wait()
        pltpu.make_async_copy(dst_hbm.at[s], dst_all, semp).wait()

        zc = z_hbm.at[c]

        def start_gather(i, b):
            pltpu.async_copy(zc.at[src_all.at[i]], rows[b], semg[b])

        def wait_gather(b):
            pltpu.make_async_copy(zc.at[src_all.at[0]], rows[b],
                                  semg[b]).wait()

        def start_scatter(i, b):
            pltpu.async_copy(rows[b], acc_sh.at[dst_all.at[i]], sems[b],
                             add=True)
            if is_deg:
                @pl.when(c == b % 2)
                def _():
                    pltpu.async_copy(ones_v, deg_sh.at[dst_all.at[i]],
                                     semd[b], add=True)

        def wait_scatter(b):
            pltpu.make_async_copy(rows[b], acc_sh.at[dst_all.at[0]],
                                  sems[b]).wait()
            if is_deg:
                @pl.when(c == b % 2)
                def _():
                    pltpu.make_async_copy(ones_v,
                                          deg_sh.at[dst_all.at[0]],
                                          semd[b]).wait()

        for b in range(NBUF):
            start_gather(b, b)

        plsc.subcore_barrier()

        ngrp = nch // NBUF

        def grp(g, carry):
            for b in range(NBUF):
                wait_gather(b)
                start_scatter(NBUF * g + b, b)
            for b in range(NBUF):
                @pl.when(NBUF * (g + 1) + b < nch)
                def _(b=b):
                    wait_scatter(b)
                    start_gather(NBUF * (g + 1) + b, b)
            return carry
        lax.fori_loop(0, ngrp, grp, 0)

        for b in range(NBUF):
            wait_scatter(b)

        plsc.subcore_barrier()

        stg = [wb_v, rows[0]]
        nwb = len(wb_chunks)
        for k, (off, sz) in enumerate(wb_chunks):
            buf = stg[k % 2]
            sem = semg[k % 2]
            if k >= 2:
                poff, psz = wb_chunks[k - 2]
                pltpu.make_async_copy(
                    buf.at[pl.ds(0, psz)],
                    out_hbm.at[c, pl.ds(row0 + poff, psz)], sem).wait()
            pltpu.sync_copy(acc_sh.at[pl.ds(row0 + off, sz)],
                            buf.at[pl.ds(0, sz)])
            pltpu.async_copy(buf.at[pl.ds(0, sz)],
                             out_hbm.at[c, pl.ds(row0 + off, sz)], sem)
        for k in range(max(0, nwb - 2), nwb):
            off, sz = wb_chunks[k]
            pltpu.make_async_copy(
                stg[k % 2].at[pl.ds(0, sz)],
                out_hbm.at[c, pl.ds(row0 + off, sz)], semg[k % 2]).wait()
        if is_deg:
            for off, sz in wb_chunks:
                pltpu.sync_copy(deg_sh.at[pl.ds(row0 + off, sz)],
                                wbd_v.at[pl.ds(0, sz)])
                pltpu.sync_copy(wbd_v.at[pl.ds(0, sz)],
                                degout_hbm.at[c, pl.ds(row0 + off, sz)])

    return pl.kernel(
        body, out_type=out_type, mesh=mesh, scratch_types=scratch,
        compiler_params=pltpu.CompilerParams(use_tc_tiling_on_sc=False))


_RB = 5000


def _row_spec(d):
    return pl.BlockSpec((_RB, d), lambda b: (b, 0))


def _half_spec(dh):
    return pl.BlockSpec((NC, _RB, dh), lambda b: (0, b, 0))


def _full_spec(shape):
    nd = len(shape)
    return pl.BlockSpec(shape, lambda b: (0,) * nd)


def _split(t, z_ref):
    dh = t.shape[-1] // 2
    z_ref[0] = t[:, :dh]
    z_ref[1] = t[:, dh:]


def _tc1_body(x_ref, w1l_ref, z1_ref):
    _split(jnp.dot(x_ref[...], w1l_ref[...], preferred_element_type=F32),
           z1_ref)


def _tc2_body(p_ref, deg_ref, x_ref, w1r_ref, b1l_ref, w2l_ref,
              a1_ref, z2_ref, inv_ref):
    agg = jnp.concatenate([p_ref[0], p_ref[1]], axis=-1)
    deg = deg_ref[0, :, :1] + deg_ref[1, :, :1]
    inv = 1.0 / jnp.maximum(deg, 1.0)
    inv_ref[...] = jnp.broadcast_to(inv, inv_ref.shape)
    r1 = jnp.dot(x_ref[...], w1r_ref[...], preferred_element_type=F32)
    h1 = jnp.maximum(agg * inv + b1l_ref[...] + r1, 0.0)
    a1_ref[...] = h1
    _split(jnp.dot(h1, w2l_ref[...], preferred_element_type=F32), z2_ref)


def _tc3_body(p_ref, inv_ref, a1_ref, x_ref, w2r_ref, b2l_ref,
              gamma_ref, beta_ref, w3l_ref, a2_ref, z3_ref):
    agg = jnp.concatenate([p_ref[0], p_ref[1]], axis=-1)
    inv = inv_ref[:, :1]
    r2 = jnp.dot(a1_ref[...], w2r_ref[...], preferred_element_type=F32)
    t = agg * inv + b2l_ref[...] + r2
    gs = gamma_ref[...] * (1.0 / jnp.sqrt(1.0 + 1e-5))
    t = t * gs + beta_ref[...]
    h2 = jnp.maximum(t, 0.0) + x_ref[...]
    a2_ref[...] = h2
    _split(jnp.dot(h2, w3l_ref[...], preferred_element_type=F32), z3_ref)


def _tc4_body(p_ref, inv_ref, a2_ref, w3r_ref, b3l_ref, logp_ref, out_ref):
    agg = jnp.concatenate([p_ref[0], p_ref[1]], axis=-1)
    inv = inv_ref[:, :1]
    r3 = jnp.dot(a2_ref[...], w3r_ref[...], preferred_element_type=F32)
    out = agg * inv + b3l_ref[...] + r3
    out_ref[...] = out
    m = jnp.max(out, axis=-1, keepdims=True)
    ex = jnp.exp(out - m)
    lse = jnp.log(jnp.sum(ex, axis=-1, keepdims=True)) + m
    logp_ref[...] = out - lse


def kernel(x, edge_index, W1l, W1r, W2l, W2r, W3l, W3r, b1l, b2l, b3l,
           gamma, beta):
    n, d = x.shape
    dh = d // NC
    e = edge_index.shape[1]
    grid = (n // _RB,)

    spmm_deg = _make_spmm(n, d, e, with_deg=True)
    spmm = _make_spmm(n, d, e, with_deg=False)

    npad = ((n + NS * 8 - 1) // (NS * 8)) * (NS * 8)
    wbz = min(npad // NS, 80)
    zeros_d = jnp.zeros((wbz, dh), F32)
    zeros16 = jnp.zeros((wbz, 8), F32)
    b1l2 = b1l.reshape(1, d)
    b2l2 = b2l.reshape(1, d)
    b3l2 = b3l.reshape(1, d)
    gamma2 = gamma.reshape(1, d)
    beta2 = beta.reshape(1, d)
    ei = edge_index.astype(jnp.int32)
    nch = e // NS // CH
    src = ei[0].reshape(NS, nch, CH)
    dst = ei[1].reshape(NS, nch, CH)

    zsplit_shape = jax.ShapeDtypeStruct((NC, npad, dh), F32)

    z1 = pl.pallas_call(
        _tc1_body, grid=grid,
        in_specs=[_row_spec(d), _full_spec((d, d))],
        out_specs=_half_spec(dh),
        out_shape=zsplit_shape,
    )(x, W1l)
    p1, deg = spmm_deg(z1, src, dst, zeros_d, zeros16)
    a1, z2, inv16 = pl.pallas_call(
        _tc2_body, grid=grid,
        in_specs=[_half_spec(dh), _half_spec(8), _row_spec(d),
                  _full_spec((d, d)), _full_spec((1, d)), _full_spec((d, d))],
        out_specs=[_row_spec(d), _half_spec(dh), _row_spec(8)],
        out_shape=[jax.ShapeDtypeStruct((n, d), F32),
                   zsplit_shape,
                   jax.ShapeDtypeStruct((n, 8), F32)],
    )(p1, deg, x, W1r, b1l2, W2l)

    (p2,) = spmm(z2, src, dst, zeros_d, zeros16)
    a2, z3 = pl.pallas_call(
        _tc3_body, grid=grid,
        in_specs=[_half_spec(dh), _row_spec(8), _row_spec(d), _row_spec(d),
                  _full_spec((d, d)), _full_spec((1, d)), _full_spec((1, d)),
                  _full_spec((1, d)), _full_spec((d, d))],
        out_specs=[_row_spec(d), _half_spec(dh)],
        out_shape=[jax.ShapeDtypeStruct((n, d), F32), zsplit_shape],
    )(p2, inv16, a1, x, W2r, b2l2, gamma2, beta2, W3l)

    (p3,) = spmm(z3, src, dst, zeros_d, zeros16)
    logp, out = pl.pallas_call(
        _tc4_body, grid=grid,
        in_specs=[_half_spec(dh), _row_spec(8), _row_spec(d),
                  _full_spec((d, d)), _full_spec((1, d))],
        out_specs=[_row_spec(d), _row_spec(d)],
        out_shape=[jax.ShapeDtypeStruct((n, d), F32),
                   jax.ShapeDtypeStruct((n, d), F32)],
    )(p3, inv16, a2, W3r, b3l2)

    return (logp, a1, a2, out)
